# baseline (speedup 1.0000x reference)
import functools

import jax
import jax.numpy as jnp
from jax import lax
from jax.experimental import pallas as pl
from jax.experimental.pallas import tpu as pltpu
from jax.experimental.pallas import tpu_sc as plsc

N = 10000
E = 320000
D = 128
NEG_SLOPE = 0.01

NPAD = 10240
DUMMY = N
NC = 2
NS = 16
EPAD = 327680
EC = EPAD // 128
ROWS_PER_TILE_DEG = EC // NS
ROWS_PER_WORKER = EC // (NC * NS)
NODE_SLICE = NPAD // NS
SUP = 8


def _rsqrt16(v):
    i = lax.bitcast_convert_type(v, jnp.int32)
    i = 0x5F3759DF - lax.shift_right_logical(i, 1)
    y = lax.bitcast_convert_type(i, jnp.float32)
    for _ in range(3):
        y = y * (1.5 - 0.5 * v * y * y)
    return y


def _deg_norm_body(src_h, dst_h, no_h, ni_h, isrc_h, hist, sbuf, dbuf, isrc, idst, ones, nbuf):
    c = lax.axis_index("c")
    s = lax.axis_index("s")

    zero16 = jnp.zeros((16,), jnp.float32)
    for k in range(NODE_SLICE // 16):
        nbuf[pl.ds(k * 16, 16)] = zero16
    pltpu.sync_copy(nbuf, hist.at[pl.ds(s * NODE_SLICE, NODE_SLICE)])
    one16 = jnp.full((16,), 1.0, jnp.float32)
    for k in range(8):
        ones[pl.ds(k * 16, 16)] = one16
    plsc.subcore_barrier()

    def step(t, carry):
        base = s * ROWS_PER_TILE_DEG + t * SUP
        pltpu.sync_copy(src_h.at[pl.ds(base, SUP)], sbuf)
        pltpu.sync_copy(dst_h.at[pl.ds(base, SUP)], dbuf)
        for j in range(SUP):
            for k in range(8):
                sv = sbuf[j, pl.ds(k * 16, 16)]
                dv = dbuf[j, pl.ds(k * 16, 16)]
                m = sv == dv
                isrc[j, pl.ds(k * 16, 16)] = jnp.where(m, DUMMY, sv)
                idst[j, pl.ds(k * 16, 16)] = jnp.where(m, DUMMY, dv)

        @pl.when(c == 0)
        def _():
            pltpu.sync_copy(isrc, isrc_h.at[pl.ds(base, SUP)])
            for j in range(SUP):
                pltpu.sync_copy(ones, hist.at[isrc.at[j]], add=True)

        @pl.when(c == 1)
        def _():
            for j in range(SUP):
                pltpu.sync_copy(ones, hist.at[idst.at[j]], add=True)

        return carry

    lax.fori_loop(0, ROWS_PER_TILE_DEG // SUP, step, 0)
    plsc.subcore_barrier()

    pltpu.sync_copy(hist.at[pl.ds(s * NODE_SLICE, NODE_SLICE)], nbuf)
    for k in range(NODE_SLICE // 16):
        v = nbuf[pl.ds(k * 16, 16)] + 1.0
        y = _rsqrt16(v)
        rows = s * NODE_SLICE + k * 16 + lax.iota(jnp.int32, 16)
        nbuf[pl.ds(k * 16, 16)] = jnp.where(rows < N, y, 0.0)

    @pl.when(c == 0)
    def _():
        pltpu.sync_copy(nbuf, no_h.at[pl.ds(s * NODE_SLICE, NODE_SLICE)])

    @pl.when(c == 1)
    def _():
        pltpu.sync_copy(nbuf, ni_h.at[pl.ds(s * NODE_SLICE, NODE_SLICE)])


_deg_norms = pl.kernel(
    _deg_norm_body,
    out_type=(
        jax.ShapeDtypeStruct((NPAD,), jnp.float32),
        jax.ShapeDtypeStruct((NPAD,), jnp.float32),
        jax.ShapeDtypeStruct((EC, 128), jnp.int32),
    ),
    mesh=plsc.VectorSubcoreMesh(core_axis_name="c", subcore_axis_name="s"),
    scratch_types=[
        pltpu.VMEM_SHARED((NPAD,), jnp.float32),
        pltpu.VMEM((SUP, 128), jnp.int32),
        pltpu.VMEM((SUP, 128), jnp.int32),
        pltpu.VMEM((SUP, 128), jnp.int32),
        pltpu.VMEM((SUP, 128), jnp.int32),
        pltpu.VMEM((128,), jnp.float32),
        pltpu.VMEM((NODE_SLICE,), jnp.float32),
    ],
)


NBUF = 2
LOOK = 1


def _edge_scatter_body(g_h, src_h, dst_h, z_h, p_h, acc, sbuf, dbuf,
                       rows0, rows1, gsem, ssem, stsem):
    c = lax.axis_index("c")
    s = lax.axis_index("s")

    pltpu.sync_copy(z_h.at[pl.ds(s * NODE_SLICE, NODE_SLICE)],
                    acc.at[pl.ds(s * NODE_SLICE, NODE_SLICE)])
    plsc.subcore_barrier()

    wid = c * NS + s
    bufs = (rows0, rows1)
    nchunk = ROWS_PER_WORKER // SUP

    base0 = wid * ROWS_PER_WORKER
    pltpu.sync_copy(src_h.at[pl.ds(base0, SUP)], sbuf.at[0])
    pltpu.sync_copy(dst_h.at[pl.ds(base0, SUP)], dbuf.at[0])

    def chunk(t, par):
        nxt = jnp.minimum(wid * ROWS_PER_WORKER + (t + 1) * SUP, EC - SUP)
        st_a = pltpu.async_copy(src_h.at[pl.ds(nxt, SUP)], sbuf.at[1 - par],
                                stsem)
        st_b = pltpu.async_copy(dst_h.at[pl.ds(nxt, SUP)], dbuf.at[1 - par],
                                stsem)
        gathers = [None] * NBUF
        scatters = [None] * NBUF
        gathers[0] = pltpu.async_copy(g_h.at[sbuf.at[par, 0]], bufs[0], gsem)
        for j in range(SUP):
            b = j % 2
            nb = (j + 1) % 2
            if j + 1 < SUP:
                if scatters[nb] is not None:
                    scatters[nb].wait()
                    scatters[nb] = None
                gathers[nb] = pltpu.async_copy(g_h.at[sbuf.at[par, j + 1]],
                                               bufs[nb], gsem)
            gathers[b].wait()
            scatters[b] = pltpu.async_copy(bufs[b], acc.at[dbuf.at[par, j]],
                                           ssem, add=True)
        st_a.wait()
        st_b.wait()
        for d in scatters:
            if d is not None:
                d.wait()

    def step(tt, carry):
        chunk(2 * tt, 0)
        chunk(2 * tt + 1, 1)
        return carry

    lax.fori_loop(0, nchunk // 2, step, 0)
    plsc.subcore_barrier()
    pltpu.sync_copy(acc.at[pl.ds(s * NODE_SLICE, NODE_SLICE)],
                    p_h.at[c, pl.ds(s * NODE_SLICE, NODE_SLICE)])


_edge_scatter = pl.kernel(
    _edge_scatter_body,
    out_type=jax.ShapeDtypeStruct((NC, NPAD, D), jnp.float32),
    mesh=plsc.VectorSubcoreMesh(core_axis_name="c", subcore_axis_name="s"),
    scratch_types=[
        pltpu.VMEM_SHARED((NPAD, D), jnp.float32),
        pltpu.VMEM((2, SUP, 128), jnp.int32),
        pltpu.VMEM((2, SUP, 128), jnp.int32),
        pltpu.VMEM((128, D), jnp.float32),
        pltpu.VMEM((128, D), jnp.float32),
        pltpu.SemaphoreType.DMA,
        pltpu.SemaphoreType.DMA,
        pltpu.SemaphoreType.DMA,
    ],
)


BLK = 1024
GRID = NPAD // BLK


def _tc_pre_body(x_ref, w_ref, no_ref, g_ref):
    g_ref[...] = jnp.dot(x_ref[...], w_ref[...],
                         preferred_element_type=jnp.float32) * no_ref[...]


_tc_pre = pl.pallas_call(
    _tc_pre_body,
    grid=(GRID,),
    in_specs=[
        pl.BlockSpec((BLK, D), lambda i: (i, 0)),
        pl.BlockSpec((D, D), lambda i: (0, 0)),
        pl.BlockSpec((BLK, 1), lambda i: (i, 0)),
    ],
    out_specs=pl.BlockSpec((BLK, D), lambda i: (i, 0)),
    out_shape=jax.ShapeDtypeStruct((NPAD, D), jnp.float32),
)


def _leaky(h):
    return jnp.where(h >= 0, h, NEG_SLOPE * h)


def _tc_mid_body(p_ref, g_ref, m_ref, ni_ref, no_ref, w_ref, b_ref, m_out, g_out):
    h = (p_ref[0] + p_ref[1] + g_ref[...]) * ni_ref[...] + b_ref[...]
    h = _leaky(h)
    m_out[...] = jnp.maximum(m_ref[...], h)
    g_out[...] = jnp.dot(h, w_ref[...],
                         preferred_element_type=jnp.float32) * no_ref[...]


_tc_mid = pl.pallas_call(
    _tc_mid_body,
    grid=(GRID,),
    in_specs=[
        pl.BlockSpec((NC, BLK, D), lambda i: (0, i, 0)),
        pl.BlockSpec((BLK, D), lambda i: (i, 0)),
        pl.BlockSpec((BLK, D), lambda i: (i, 0)),
        pl.BlockSpec((BLK, 1), lambda i: (i, 0)),
        pl.BlockSpec((BLK, 1), lambda i: (i, 0)),
        pl.BlockSpec((D, D), lambda i: (0, 0)),
        pl.BlockSpec((1, D), lambda i: (0, 0)),
    ],
    out_specs=[
        pl.BlockSpec((BLK, D), lambda i: (i, 0)),
        pl.BlockSpec((BLK, D), lambda i: (i, 0)),
    ],
    out_shape=[
        jax.ShapeDtypeStruct((NPAD, D), jnp.float32),
        jax.ShapeDtypeStruct((NPAD, D), jnp.float32),
    ],
)


def _tc_fin_body(p_ref, g_ref, m_ref, ni_ref, b_ref, o_ref):
    h = (p_ref[0] + p_ref[1] + g_ref[...]) * ni_ref[...] + b_ref[...]
    o_ref[...] = jnp.maximum(m_ref[...], _leaky(h))


_tc_fin = pl.pallas_call(
    _tc_fin_body,
    grid=(GRID,),
    in_specs=[
        pl.BlockSpec((NC, BLK, D), lambda i: (0, i, 0)),
        pl.BlockSpec((BLK, D), lambda i: (i, 0)),
        pl.BlockSpec((BLK, D), lambda i: (i, 0)),
        pl.BlockSpec((BLK, 1), lambda i: (i, 0)),
        pl.BlockSpec((1, D), lambda i: (0, 0)),
    ],
    out_specs=pl.BlockSpec((BLK, D), lambda i: (i, 0)),
    out_shape=jax.ShapeDtypeStruct((NPAD, D), jnp.float32),
)


@functools.partial(jax.jit, static_argnums=())
def kernel(x, edge_index, W0, b0, W1, b1, W2, b2, W3, b3):
    src = edge_index[0]
    dst = edge_index[1]
    npad_e = EPAD - E
    pad_iota = jnp.arange(npad_e, dtype=jnp.int32)
    src_p = jnp.concatenate([src, N + pad_iota % 240]).reshape(EC, 128)
    dst_p = jnp.concatenate([dst, N + (pad_iota + 120) % 240]).reshape(EC, 128)

    x_p = jnp.pad(x, ((0, NPAD - N), (0, 0)))
    zeros2d = jnp.zeros((NPAD, D), jnp.float32)

    norm_out, norm_in, isrc = _deg_norms(src_p, dst_p)
    no_col = norm_out.reshape(NPAD, 1)
    ni_col = norm_in.reshape(NPAD, 1)

    b0r = b0.reshape(1, D)
    b1r = b1.reshape(1, D)
    b2r = b2.reshape(1, D)
    b3r = b3.reshape(1, D)

    g = _tc_pre(x_p, W0, no_col)
    p = _edge_scatter(g, isrc, dst_p, zeros2d)
    m, g = _tc_mid(p, g, x_p, ni_col, no_col, W1, b0r)
    p = _edge_scatter(g, isrc, dst_p, zeros2d)
    m, g = _tc_mid(p, g, m, ni_col, no_col, W2, b1r)
    p = _edge_scatter(g, isrc, dst_p, zeros2d)
    m, g = _tc_mid(p, g, m, ni_col, no_col, W3, b2r)
    p = _edge_scatter(g, isrc, dst_p, zeros2d)
    out = _tc_fin(p, g, m, ni_col, b3r)
    return out[:N]


# speedup vs baseline: 1.0510x; 1.0510x over previous
import functools

import jax
import jax.numpy as jnp
from jax import lax
from jax.experimental import pallas as pl
from jax.experimental.pallas import tpu as pltpu
from jax.experimental.pallas import tpu_sc as plsc

N = 10000
E = 320000
D = 128
NEG_SLOPE = 0.01

NPAD = 10240
DUMMY = N
NC = 2
NS = 16
EPAD = 327680
EC = EPAD // 128
ROWS_PER_TILE_DEG = EC // NS
ROWS_PER_WORKER = EC // (NC * NS)
NODE_SLICE = NPAD // NS
SUP = 8


def _rsqrt16(v):
    i = lax.bitcast_convert_type(v, jnp.int32)
    i = 0x5F3759DF - lax.shift_right_logical(i, 1)
    y = lax.bitcast_convert_type(i, jnp.float32)
    for _ in range(3):
        y = y * (1.5 - 0.5 * v * y * y)
    return y


def _deg_norm_body(src_h, dst_h, no_h, ni_h, isrc_h, hist, sbuf, dbuf,
                   isrc0, isrc1, ibuf0, ibuf1, ones, nbuf, stsem, hsem, wsem):
    c = lax.axis_index("c")
    s = lax.axis_index("s")

    zero16 = jnp.zeros((16,), jnp.float32)
    for k in range(NODE_SLICE // 16):
        nbuf[pl.ds(k * 16, 16)] = zero16
    pltpu.sync_copy(nbuf, hist.at[pl.ds(s * NODE_SLICE, NODE_SLICE)])
    one16 = jnp.full((16,), 1.0, jnp.float32)
    for k in range(8):
        ones[pl.ds(k * 16, 16)] = one16
    plsc.subcore_barrier()

    base0 = s * ROWS_PER_TILE_DEG
    pltpu.sync_copy(src_h.at[pl.ds(base0, SUP)], sbuf.at[0])
    pltpu.sync_copy(dst_h.at[pl.ds(base0, SUP)], dbuf.at[0])

    def chunk(t, par):
        nxt = jnp.minimum(s * ROWS_PER_TILE_DEG + (t + 1) * SUP,
                          EC - SUP)
        st_a = pltpu.async_copy(src_h.at[pl.ds(nxt, SUP)], sbuf.at[1 - par],
                                stsem)
        st_b = pltpu.async_copy(dst_h.at[pl.ds(nxt, SUP)], dbuf.at[1 - par],
                                stsem)
        isrc = (isrc0, isrc1)[par]
        ibuf = (ibuf0, ibuf1)[par]
        cf = jnp.zeros((16,), jnp.int32) + c
        for j in range(SUP):
            for k in range(8):
                sv = sbuf[par, j, pl.ds(k * 16, 16)]
                dv = dbuf[par, j, pl.ds(k * 16, 16)]
                m = sv == dv
                svm = jnp.where(m, DUMMY, sv)
                dvm = jnp.where(m, DUMMY, dv)
                isrc[j, pl.ds(k * 16, 16)] = svm
                ibuf[j, pl.ds(k * 16, 16)] = svm + (dvm - svm) * cf

        @pl.when(c == 0)
        def _():
            base = s * ROWS_PER_TILE_DEG + t * SUP
            pltpu.sync_copy(isrc, isrc_h.at[pl.ds(base, SUP)])

        pend = []
        for j in range(SUP):
            pend.append(pltpu.async_copy(ones, hist.at[ibuf.at[j]],
                                         hsem, add=True))
        st_a.wait()
        st_b.wait()
        return pend

    def step(tt, carry):
        pend_a = chunk(2 * tt, 0)
        pend_b = chunk(2 * tt + 1, 1)
        for d in pend_a + pend_b:
            d.wait()
        return carry

    lax.fori_loop(0, ROWS_PER_TILE_DEG // SUP // 2, step, 0)
    plsc.subcore_barrier()

    pltpu.sync_copy(hist.at[pl.ds(s * NODE_SLICE, NODE_SLICE)], nbuf)
    for k in range(NODE_SLICE // 16):
        v = nbuf[pl.ds(k * 16, 16)] + 1.0
        y = _rsqrt16(v)
        rows = s * NODE_SLICE + k * 16 + lax.iota(jnp.int32, 16)
        nbuf[pl.ds(k * 16, 16)] = jnp.where(rows < N, y, 0.0)

    @pl.when(c == 0)
    def _():
        pltpu.sync_copy(nbuf, no_h.at[pl.ds(s * NODE_SLICE, NODE_SLICE)])

    @pl.when(c == 1)
    def _():
        pltpu.sync_copy(nbuf, ni_h.at[pl.ds(s * NODE_SLICE, NODE_SLICE)])


_deg_norms = pl.kernel(
    _deg_norm_body,
    out_type=(
        jax.ShapeDtypeStruct((NPAD,), jnp.float32),
        jax.ShapeDtypeStruct((NPAD,), jnp.float32),
        jax.ShapeDtypeStruct((EC, 128), jnp.int32),
    ),
    mesh=plsc.VectorSubcoreMesh(core_axis_name="c", subcore_axis_name="s"),
    scratch_types=[
        pltpu.VMEM_SHARED((NPAD,), jnp.float32),
        pltpu.VMEM((2, SUP, 128), jnp.int32),
        pltpu.VMEM((2, SUP, 128), jnp.int32),
        pltpu.VMEM((SUP, 128), jnp.int32),
        pltpu.VMEM((SUP, 128), jnp.int32),
        pltpu.VMEM((SUP, 128), jnp.int32),
        pltpu.VMEM((SUP, 128), jnp.int32),
        pltpu.VMEM((128,), jnp.float32),
        pltpu.VMEM((NODE_SLICE,), jnp.float32),
        pltpu.SemaphoreType.DMA,
        pltpu.SemaphoreType.DMA,
        pltpu.SemaphoreType.DMA,
    ],
)


NBUF = 2
LOOK = 1


def _edge_scatter_body(g_h, src_h, dst_h, z_h, p_h, acc, sbuf, dbuf,
                       rows0, rows1, gsem, ssem, stsem):
    c = lax.axis_index("c")
    s = lax.axis_index("s")

    pltpu.sync_copy(z_h.at[pl.ds(s * NODE_SLICE, NODE_SLICE)],
                    acc.at[pl.ds(s * NODE_SLICE, NODE_SLICE)])
    plsc.subcore_barrier()

    wid = c * NS + s
    bufs = (rows0, rows1)
    nchunk = ROWS_PER_WORKER // SUP

    base0 = wid * ROWS_PER_WORKER
    pltpu.sync_copy(src_h.at[pl.ds(base0, SUP)], sbuf.at[0])
    pltpu.sync_copy(dst_h.at[pl.ds(base0, SUP)], dbuf.at[0])

    def chunk(t, par):
        nxt = jnp.minimum(wid * ROWS_PER_WORKER + (t + 1) * SUP, EC - SUP)
        st_a = pltpu.async_copy(src_h.at[pl.ds(nxt, SUP)], sbuf.at[1 - par],
                                stsem)
        st_b = pltpu.async_copy(dst_h.at[pl.ds(nxt, SUP)], dbuf.at[1 - par],
                                stsem)
        gathers = [None] * NBUF
        scatters = [None] * NBUF
        gathers[0] = pltpu.async_copy(g_h.at[sbuf.at[par, 0]], bufs[0], gsem)
        for j in range(SUP):
            b = j % 2
            nb = (j + 1) % 2
            if j + 1 < SUP:
                if scatters[nb] is not None:
                    scatters[nb].wait()
                    scatters[nb] = None
                gathers[nb] = pltpu.async_copy(g_h.at[sbuf.at[par, j + 1]],
                                               bufs[nb], gsem)
            gathers[b].wait()
            scatters[b] = pltpu.async_copy(bufs[b], acc.at[dbuf.at[par, j]],
                                           ssem, add=True)
        st_a.wait()
        st_b.wait()
        for d in scatters:
            if d is not None:
                d.wait()

    def step(tt, carry):
        chunk(2 * tt, 0)
        chunk(2 * tt + 1, 1)
        return carry

    lax.fori_loop(0, nchunk // 2, step, 0)
    plsc.subcore_barrier()
    pltpu.sync_copy(acc.at[pl.ds(s * NODE_SLICE, NODE_SLICE)],
                    p_h.at[c, pl.ds(s * NODE_SLICE, NODE_SLICE)])


_edge_scatter = pl.kernel(
    _edge_scatter_body,
    out_type=jax.ShapeDtypeStruct((NC, NPAD, D), jnp.float32),
    mesh=plsc.VectorSubcoreMesh(core_axis_name="c", subcore_axis_name="s"),
    scratch_types=[
        pltpu.VMEM_SHARED((NPAD, D), jnp.float32),
        pltpu.VMEM((2, SUP, 128), jnp.int32),
        pltpu.VMEM((2, SUP, 128), jnp.int32),
        pltpu.VMEM((128, D), jnp.float32),
        pltpu.VMEM((128, D), jnp.float32),
        pltpu.SemaphoreType.DMA,
        pltpu.SemaphoreType.DMA,
        pltpu.SemaphoreType.DMA,
    ],
)


BLK = 1024
GRID = NPAD // BLK


def _tc_pre_body(x_ref, w_ref, no_ref, g_ref):
    g_ref[...] = jnp.dot(x_ref[...], w_ref[...],
                         preferred_element_type=jnp.float32) * no_ref[...]


_tc_pre = pl.pallas_call(
    _tc_pre_body,
    grid=(GRID,),
    in_specs=[
        pl.BlockSpec((BLK, D), lambda i: (i, 0)),
        pl.BlockSpec((D, D), lambda i: (0, 0)),
        pl.BlockSpec((BLK, 1), lambda i: (i, 0)),
    ],
    out_specs=pl.BlockSpec((BLK, D), lambda i: (i, 0)),
    out_shape=jax.ShapeDtypeStruct((NPAD, D), jnp.float32),
)


def _leaky(h):
    return jnp.where(h >= 0, h, NEG_SLOPE * h)


def _tc_mid_body(p_ref, g_ref, m_ref, ni_ref, no_ref, w_ref, b_ref, m_out, g_out):
    h = (p_ref[0] + p_ref[1] + g_ref[...]) * ni_ref[...] + b_ref[...]
    h = _leaky(h)
    m_out[...] = jnp.maximum(m_ref[...], h)
    g_out[...] = jnp.dot(h, w_ref[...],
                         preferred_element_type=jnp.float32) * no_ref[...]


_tc_mid = pl.pallas_call(
    _tc_mid_body,
    grid=(GRID,),
    in_specs=[
        pl.BlockSpec((NC, BLK, D), lambda i: (0, i, 0)),
        pl.BlockSpec((BLK, D), lambda i: (i, 0)),
        pl.BlockSpec((BLK, D), lambda i: (i, 0)),
        pl.BlockSpec((BLK, 1), lambda i: (i, 0)),
        pl.BlockSpec((BLK, 1), lambda i: (i, 0)),
        pl.BlockSpec((D, D), lambda i: (0, 0)),
        pl.BlockSpec((1, D), lambda i: (0, 0)),
    ],
    out_specs=[
        pl.BlockSpec((BLK, D), lambda i: (i, 0)),
        pl.BlockSpec((BLK, D), lambda i: (i, 0)),
    ],
    out_shape=[
        jax.ShapeDtypeStruct((NPAD, D), jnp.float32),
        jax.ShapeDtypeStruct((NPAD, D), jnp.float32),
    ],
)


def _tc_fin_body(p_ref, g_ref, m_ref, ni_ref, b_ref, o_ref):
    h = (p_ref[0] + p_ref[1] + g_ref[...]) * ni_ref[...] + b_ref[...]
    o_ref[...] = jnp.maximum(m_ref[...], _leaky(h))


BLKF = 1000

_tc_fin = pl.pallas_call(
    _tc_fin_body,
    grid=(N // BLKF,),
    in_specs=[
        pl.BlockSpec((NC, BLKF, D), lambda i: (0, i, 0)),
        pl.BlockSpec((BLKF, D), lambda i: (i, 0)),
        pl.BlockSpec((BLKF, D), lambda i: (i, 0)),
        pl.BlockSpec((BLKF, 1), lambda i: (i, 0)),
        pl.BlockSpec((1, D), lambda i: (0, 0)),
    ],
    out_specs=pl.BlockSpec((BLKF, D), lambda i: (i, 0)),
    out_shape=jax.ShapeDtypeStruct((N, D), jnp.float32),
)


@functools.partial(jax.jit, static_argnums=())
def kernel(x, edge_index, W0, b0, W1, b1, W2, b2, W3, b3):
    src = edge_index[0]
    dst = edge_index[1]
    npad_e = EPAD - E
    pad_iota = jnp.arange(npad_e, dtype=jnp.int32)
    src_p = jnp.concatenate([src, N + pad_iota % 240]).reshape(EC, 128)
    dst_p = jnp.concatenate([dst, N + (pad_iota + 120) % 240]).reshape(EC, 128)

    x_p = jnp.pad(x, ((0, NPAD - N), (0, 0)))
    zeros2d = jnp.zeros((NPAD, D), jnp.float32)

    norm_out, norm_in, isrc = _deg_norms(src_p, dst_p)
    no_col = norm_out.reshape(NPAD, 1)
    ni_col = norm_in.reshape(NPAD, 1)

    b0r = b0.reshape(1, D)
    b1r = b1.reshape(1, D)
    b2r = b2.reshape(1, D)
    b3r = b3.reshape(1, D)

    g = _tc_pre(x_p, W0, no_col)
    p = _edge_scatter(g, isrc, dst_p, zeros2d)
    m, g = _tc_mid(p, g, x_p, ni_col, no_col, W1, b0r)
    p = _edge_scatter(g, isrc, dst_p, zeros2d)
    m, g = _tc_mid(p, g, m, ni_col, no_col, W2, b1r)
    p = _edge_scatter(g, isrc, dst_p, zeros2d)
    m, g = _tc_mid(p, g, m, ni_col, no_col, W3, b2r)
    p = _edge_scatter(g, isrc, dst_p, zeros2d)
    return _tc_fin(p, g, m, ni_col, b3r)


# speedup vs baseline: 1.2418x; 1.1815x over previous
import functools

import jax
import jax.numpy as jnp
from jax import lax
from jax.experimental import pallas as pl
from jax.experimental.pallas import tpu as pltpu
from jax.experimental.pallas import tpu_sc as plsc

N = 10000
E = 320000
D = 128
NEG_SLOPE = 0.01

NPAD = 10240
DUMMY = N
NC = 2
NS = 16
EPAD = 327680
EC = EPAD // 128
ROWS_PER_TILE_DEG = EC // NS
ROWS_PER_WORKER = EC // (NC * NS)
NODE_SLICE = NPAD // NS
SUP = 8
SUPE = 16


def _rsqrt16(v):
    i = lax.bitcast_convert_type(v, jnp.int32)
    i = 0x5F3759DF - lax.shift_right_logical(i, 1)
    y = lax.bitcast_convert_type(i, jnp.float32)
    for _ in range(3):
        y = y * (1.5 - 0.5 * v * y * y)
    return y


def _deg_norm_body(src_h, dst_h, no_h, ni_h, isrc_h, hist, sbuf, dbuf,
                   isrc0, isrc1, ibuf0, ibuf1, ones, nbuf, stsem, hsem, wsem):
    c = lax.axis_index("c")
    s = lax.axis_index("s")

    zero16 = jnp.zeros((16,), jnp.float32)
    for k in range(NODE_SLICE // 16):
        nbuf[pl.ds(k * 16, 16)] = zero16
    pltpu.sync_copy(nbuf, hist.at[pl.ds(s * NODE_SLICE, NODE_SLICE)])
    one16 = jnp.full((16,), 1.0, jnp.float32)
    for k in range(8):
        ones[pl.ds(k * 16, 16)] = one16
    plsc.subcore_barrier()

    base0 = s * ROWS_PER_TILE_DEG
    pltpu.sync_copy(src_h.at[pl.ds(base0, SUP)], sbuf.at[0])
    pltpu.sync_copy(dst_h.at[pl.ds(base0, SUP)], dbuf.at[0])

    def chunk(t, par):
        nxt = jnp.minimum(s * ROWS_PER_TILE_DEG + (t + 1) * SUP,
                          EC - SUP)
        st_a = pltpu.async_copy(src_h.at[pl.ds(nxt, SUP)], sbuf.at[1 - par],
                                stsem)
        st_b = pltpu.async_copy(dst_h.at[pl.ds(nxt, SUP)], dbuf.at[1 - par],
                                stsem)
        isrc = (isrc0, isrc1)[par]
        ibuf = (ibuf0, ibuf1)[par]
        cf = jnp.zeros((16,), jnp.int32) + c
        for j in range(SUP):
            for k in range(8):
                sv = sbuf[par, j, pl.ds(k * 16, 16)]
                dv = dbuf[par, j, pl.ds(k * 16, 16)]
                m = sv == dv
                svm = jnp.where(m, DUMMY, sv)
                dvm = jnp.where(m, DUMMY, dv)
                isrc[j, pl.ds(k * 16, 16)] = svm
                ibuf[j, pl.ds(k * 16, 16)] = svm + (dvm - svm) * cf

        @pl.when(c == 0)
        def _():
            base = s * ROWS_PER_TILE_DEG + t * SUP
            pltpu.sync_copy(isrc, isrc_h.at[pl.ds(base, SUP)])

        pend = []
        for j in range(SUP):
            pend.append(pltpu.async_copy(ones, hist.at[ibuf.at[j]],
                                         hsem, add=True))
        st_a.wait()
        st_b.wait()
        return pend

    def step(tt, carry):
        pend_a = chunk(2 * tt, 0)
        pend_b = chunk(2 * tt + 1, 1)
        for d in pend_a + pend_b:
            d.wait()
        return carry

    lax.fori_loop(0, ROWS_PER_TILE_DEG // SUP // 2, step, 0)
    plsc.subcore_barrier()

    pltpu.sync_copy(hist.at[pl.ds(s * NODE_SLICE, NODE_SLICE)], nbuf)
    for k in range(NODE_SLICE // 16):
        v = nbuf[pl.ds(k * 16, 16)] + 1.0
        y = _rsqrt16(v)
        rows = s * NODE_SLICE + k * 16 + lax.iota(jnp.int32, 16)
        nbuf[pl.ds(k * 16, 16)] = jnp.where(rows < N, y, 0.0)

    @pl.when(c == 0)
    def _():
        pltpu.sync_copy(nbuf, no_h.at[pl.ds(s * NODE_SLICE, NODE_SLICE)])

    @pl.when(c == 1)
    def _():
        pltpu.sync_copy(nbuf, ni_h.at[pl.ds(s * NODE_SLICE, NODE_SLICE)])


_deg_norms = pl.kernel(
    _deg_norm_body,
    out_type=(
        jax.ShapeDtypeStruct((NPAD,), jnp.float32),
        jax.ShapeDtypeStruct((NPAD,), jnp.float32),
        jax.ShapeDtypeStruct((EC, 128), jnp.int32),
    ),
    mesh=plsc.VectorSubcoreMesh(core_axis_name="c", subcore_axis_name="s"),
    scratch_types=[
        pltpu.VMEM_SHARED((NPAD,), jnp.float32),
        pltpu.VMEM((2, SUP, 128), jnp.int32),
        pltpu.VMEM((2, SUP, 128), jnp.int32),
        pltpu.VMEM((SUP, 128), jnp.int32),
        pltpu.VMEM((SUP, 128), jnp.int32),
        pltpu.VMEM((SUP, 128), jnp.int32),
        pltpu.VMEM((SUP, 128), jnp.int32),
        pltpu.VMEM((128,), jnp.float32),
        pltpu.VMEM((NODE_SLICE,), jnp.float32),
        pltpu.SemaphoreType.DMA,
        pltpu.SemaphoreType.DMA,
        pltpu.SemaphoreType.DMA,
    ],
)


NBUF = 2
LOOK = 1


def _edge_scatter_body(g_h, src_h, dst_h, z_h, p_h, acc, sbuf, dbuf,
                       rows0, rows1, gsem, ssem, stsem):
    c = lax.axis_index("c")
    s = lax.axis_index("s")

    pltpu.sync_copy(z_h.at[pl.ds(s * NODE_SLICE, NODE_SLICE)],
                    acc.at[pl.ds(s * NODE_SLICE, NODE_SLICE)])
    plsc.subcore_barrier()

    wid = c * NS + s
    bufs = (rows0, rows1)
    nchunk = ROWS_PER_WORKER // SUPE

    base0 = wid * ROWS_PER_WORKER
    pltpu.sync_copy(src_h.at[pl.ds(base0, SUPE)], sbuf.at[0])
    pltpu.sync_copy(dst_h.at[pl.ds(base0, SUPE)], dbuf.at[0])

    def chunk(t, par):
        nxt = jnp.minimum(wid * ROWS_PER_WORKER + (t + 1) * SUPE, EC - SUPE)
        st_a = pltpu.async_copy(src_h.at[pl.ds(nxt, SUPE)], sbuf.at[1 - par],
                                stsem)
        st_b = pltpu.async_copy(dst_h.at[pl.ds(nxt, SUPE)], dbuf.at[1 - par],
                                stsem)
        gathers = [None] * NBUF
        scatters = [None] * NBUF
        gathers[0] = pltpu.async_copy(g_h.at[sbuf.at[par, 0]], bufs[0], gsem)
        for j in range(SUPE):
            b = j % 2
            nb = (j + 1) % 2
            if j + 1 < SUPE:
                if scatters[nb] is not None:
                    scatters[nb].wait()
                    scatters[nb] = None
                gathers[nb] = pltpu.async_copy(g_h.at[sbuf.at[par, j + 1]],
                                               bufs[nb], gsem)
            gathers[b].wait()
            scatters[b] = pltpu.async_copy(bufs[b], acc.at[dbuf.at[par, j]],
                                           ssem, add=True)
        st_a.wait()
        st_b.wait()
        for d in scatters:
            if d is not None:
                d.wait()

    def step(tt, carry):
        chunk(2 * tt, 0)
        chunk(2 * tt + 1, 1)
        return carry

    lax.fori_loop(0, nchunk // 2, step, 0)
    plsc.subcore_barrier()
    pltpu.sync_copy(acc.at[pl.ds(s * NODE_SLICE, NODE_SLICE)],
                    p_h.at[c, pl.ds(s * NODE_SLICE, NODE_SLICE)])


_edge_scatter = pl.kernel(
    _edge_scatter_body,
    out_type=jax.ShapeDtypeStruct((NC, NPAD, D), jnp.float32),
    mesh=plsc.VectorSubcoreMesh(core_axis_name="c", subcore_axis_name="s"),
    scratch_types=[
        pltpu.VMEM_SHARED((NPAD, D), jnp.float32),
        pltpu.VMEM((2, SUPE, 128), jnp.int32),
        pltpu.VMEM((2, SUPE, 128), jnp.int32),
        pltpu.VMEM((128, D), jnp.float32),
        pltpu.VMEM((128, D), jnp.float32),
        pltpu.SemaphoreType.DMA,
        pltpu.SemaphoreType.DMA,
        pltpu.SemaphoreType.DMA,
    ],
)


BLK = 1024
GRID = NPAD // BLK


def _tc_pre_body(x_ref, w_ref, no_ref, g_ref):
    g_ref[...] = jnp.dot(x_ref[...], w_ref[...],
                         preferred_element_type=jnp.float32) * no_ref[...]


_tc_pre = pl.pallas_call(
    _tc_pre_body,
    grid=(GRID,),
    in_specs=[
        pl.BlockSpec((BLK, D), lambda i: (i, 0)),
        pl.BlockSpec((D, D), lambda i: (0, 0)),
        pl.BlockSpec((BLK, 1), lambda i: (i, 0)),
    ],
    out_specs=pl.BlockSpec((BLK, D), lambda i: (i, 0)),
    out_shape=jax.ShapeDtypeStruct((NPAD, D), jnp.float32),
)


def _leaky(h):
    return jnp.where(h >= 0, h, NEG_SLOPE * h)


def _tc_mid_body(p_ref, g_ref, m_ref, ni_ref, no_ref, w_ref, b_ref, m_out, g_out):
    h = (p_ref[0] + p_ref[1] + g_ref[...]) * ni_ref[...] + b_ref[...]
    h = _leaky(h)
    m_out[...] = jnp.maximum(m_ref[...], h)
    g_out[...] = jnp.dot(h, w_ref[...],
                         preferred_element_type=jnp.float32) * no_ref[...]


_tc_mid = pl.pallas_call(
    _tc_mid_body,
    grid=(GRID,),
    in_specs=[
        pl.BlockSpec((NC, BLK, D), lambda i: (0, i, 0)),
        pl.BlockSpec((BLK, D), lambda i: (i, 0)),
        pl.BlockSpec((BLK, D), lambda i: (i, 0)),
        pl.BlockSpec((BLK, 1), lambda i: (i, 0)),
        pl.BlockSpec((BLK, 1), lambda i: (i, 0)),
        pl.BlockSpec((D, D), lambda i: (0, 0)),
        pl.BlockSpec((1, D), lambda i: (0, 0)),
    ],
    out_specs=[
        pl.BlockSpec((BLK, D), lambda i: (i, 0)),
        pl.BlockSpec((BLK, D), lambda i: (i, 0)),
    ],
    out_shape=[
        jax.ShapeDtypeStruct((NPAD, D), jnp.float32),
        jax.ShapeDtypeStruct((NPAD, D), jnp.float32),
    ],
)


def _tc_fin_body(p_ref, g_ref, m_ref, ni_ref, b_ref, o_ref):
    h = (p_ref[0] + p_ref[1] + g_ref[...]) * ni_ref[...] + b_ref[...]
    o_ref[...] = jnp.maximum(m_ref[...], _leaky(h))


BLKF = 1000

_tc_fin = pl.pallas_call(
    _tc_fin_body,
    grid=(N // BLKF,),
    in_specs=[
        pl.BlockSpec((NC, BLKF, D), lambda i: (0, i, 0)),
        pl.BlockSpec((BLKF, D), lambda i: (i, 0)),
        pl.BlockSpec((BLKF, D), lambda i: (i, 0)),
        pl.BlockSpec((BLKF, 1), lambda i: (i, 0)),
        pl.BlockSpec((1, D), lambda i: (0, 0)),
    ],
    out_specs=pl.BlockSpec((BLKF, D), lambda i: (i, 0)),
    out_shape=jax.ShapeDtypeStruct((N, D), jnp.float32),
)


@functools.partial(jax.jit, static_argnums=())
def kernel(x, edge_index, W0, b0, W1, b1, W2, b2, W3, b3):
    src = edge_index[0]
    dst = edge_index[1]
    npad_e = EPAD - E
    pad_iota = jnp.arange(npad_e, dtype=jnp.int32)
    src_p = jnp.concatenate([src, N + pad_iota % 240]).reshape(EC, 128)
    dst_p = jnp.concatenate([dst, N + (pad_iota + 120) % 240]).reshape(EC, 128)

    x_p = jnp.pad(x, ((0, NPAD - N), (0, 0)))
    zeros2d = jnp.zeros((NPAD, D), jnp.float32)

    norm_out, norm_in, isrc = _deg_norms(src_p, dst_p)
    no_col = norm_out.reshape(NPAD, 1)
    ni_col = norm_in.reshape(NPAD, 1)

    b0r = b0.reshape(1, D)
    b1r = b1.reshape(1, D)
    b2r = b2.reshape(1, D)
    b3r = b3.reshape(1, D)

    g = _tc_pre(x_p, W0, no_col)
    p = _edge_scatter(g, isrc, dst_p, zeros2d)
    m, g = _tc_mid(p, g, x_p, ni_col, no_col, W1, b0r)
    p = _edge_scatter(g, isrc, dst_p, zeros2d)
    m, g = _tc_mid(p, g, m, ni_col, no_col, W2, b1r)
    p = _edge_scatter(g, isrc, dst_p, zeros2d)
    m, g = _tc_mid(p, g, m, ni_col, no_col, W3, b2r)
    p = _edge_scatter(g, isrc, dst_p, zeros2d)
    return _tc_fin(p, g, m, ni_col, b3r)
